# baseline (speedup 1.0000x reference)
import functools

import jax
import jax.numpy as jnp
from jax import lax
from jax.experimental import pallas as pl
from jax.experimental.pallas import tpu as pltpu
from jax.experimental.pallas import tpu_sc as plsc

NC, NS, L = 2, 16, 16
NW = NC * NS
B, N = 128, 32768
ROWS_PER_W = B // NW
GRP = 8
NGRP = N // (GRP * L)
KTOP = 10
CAND = 16


def _splat(x_scalar):
    return jnp.full((L,), x_scalar, jnp.float32)


def _sc_body(scores_hbm, out_hbm, row_v, out_v, pad_v):
    cid = lax.axis_index("c")
    sid = lax.axis_index("s")
    wid = sid * NC + cid
    inf = jnp.float32(jnp.inf)

    for rr in range(ROWS_PER_W):
        row = wid * ROWS_PER_W + rr
        pltpu.sync_copy(scores_hbm.at[row], row_v)

        def grp_body(g, carry):
            C, thr = carry
            base = g * (GRP * L)
            vs = [row_v[pl.ds(base + c * L, L)] for c in range(GRP)]
            thrv = _splat(thr)
            any_gt = vs[0] > thrv
            for c in range(1, GRP):
                any_gt = jnp.logical_or(any_gt, vs[c] > thrv)
            pred = jnp.any(any_gt)

            def merge(carry):
                C, _ = carry
                for c in range(GRP):
                    v_desc, _ = plsc.sort_key_val(vs[c], vs[c],
                                                  descending=True)
                    C = jnp.maximum(C, v_desc)
                    C, _ = plsc.sort_key_val(C, C)
                return C, jnp.min(C)

            def keep(carry):
                return carry

            return lax.cond(pred, merge, keep, (C, thr))

        C0 = _splat(-inf)
        C, _ = lax.fori_loop(0, NGRP, grp_body, (C0, -inf))

        pad_v[pl.ds(L, L)] = _splat(inf)
        pad_v[pl.ds(0, L)] = C
        t10 = jnp.min(pad_v[pl.ds(CAND - KTOP, L)])
        t10v = _splat(t10)
        g_cnt = jnp.sum((C > t10v).astype(jnp.int32))
        rtake = KTOP - g_cnt

        def wgrp_body(g, e_seen):
            base = g * (GRP * L)
            vs = [row_v[pl.ds(base + c * L, L)] for c in range(GRP)]
            eqs = [v == t10v for v in vs]
            any_eq = eqs[0]
            for c in range(1, GRP):
                any_eq = jnp.logical_or(any_eq, eqs[c])
            pred = jnp.any(any_eq)

            def rare(e_seen):
                e = e_seen
                for c in range(GRP):
                    eq_i = eqs[c].astype(jnp.int32)
                    prefix = plsc.cumsum(eq_i) + e
                    sel = jnp.logical_and(eqs[c], prefix <= rtake)
                    m = jnp.logical_or(vs[c] > t10v, sel)
                    out_v[pl.ds(base + c * L, L)] = jnp.where(m, 1.0, 0.0)
                    e = e + jnp.sum(eq_i)
                return e

            def common(e_seen):
                for c in range(GRP):
                    m = vs[c] > t10v
                    out_v[pl.ds(base + c * L, L)] = jnp.where(m, 1.0, 0.0)
                return e_seen

            return lax.cond(pred, rare, common, e_seen)

        lax.fori_loop(0, NGRP, wgrp_body, jnp.int32(0))

        pltpu.sync_copy(out_v, out_hbm.at[row])


@jax.jit
def kernel(scores):
    mesh = plsc.VectorSubcoreMesh(core_axis_name="c", subcore_axis_name="s")
    sc_call = functools.partial(
        pl.kernel,
        out_type=jax.ShapeDtypeStruct((B, N), jnp.float32),
        mesh=mesh,
        compiler_params=pltpu.CompilerParams(needs_layout_passes=False),
        scratch_types=[
            pltpu.VMEM((N,), jnp.float32),
            pltpu.VMEM((N,), jnp.float32),
            pltpu.VMEM((2 * L,), jnp.float32),
        ],
    )(_sc_body)
    return sc_call(scores)


# speedup vs baseline: 2.7362x; 2.7362x over previous
import functools

import jax
import jax.numpy as jnp
from jax import lax
from jax.experimental import pallas as pl
from jax.experimental.pallas import tpu as pltpu
from jax.experimental.pallas import tpu_sc as plsc

NC, NS, L = 2, 16, 16
NW = NC * NS
B, N = 128, 32768
ROWS_PER_W = B // NW
CHUNKS = N // L
GRP = 16
NGRP = CHUNKS // GRP
UZ = 8
KTOP = 10


def _splat(x, dtype=jnp.float32):
    return jnp.full((L,), x, dtype)


def _sc_body(scores_hbm, out_hbm, row_a, row_b, zero_v, sem, _):
    cid = lax.axis_index("c")
    sid = lax.axis_index("s")
    wid = sid * NC + cid
    inf = jnp.float32(jnp.inf)
    zeros_f = _splat(0.0)
    ones_f = _splat(1.0)
    lane_iota = lax.broadcasted_iota(jnp.int32, (L,), 0)

    def zfill(i, _):
        for u in range(UZ):
            zero_v[pl.ds((i * UZ + u) * L, L)] = zeros_f
        return 0

    lax.fori_loop(0, CHUNKS // UZ, zfill, 0)

    row0 = wid * ROWS_PER_W
    bufs = (row_a, row_b)
    pltpu.async_copy(scores_hbm.at[row0], row_a, sem).wait()

    for rr in range(ROWS_PER_W):
        row = row0 + rr
        row_v = bufs[rr % 2]
        if rr + 1 < ROWS_PER_W:
            nxt = pltpu.async_copy(scores_hbm.at[row + 1], bufs[(rr + 1) % 2],
                                   sem)

        def amax_body(i, vs):
            out = []
            for u in range(UZ):
                v = row_v[pl.ds((i * UZ + u) * L, L)]
                out.append(jnp.maximum(vs[u], v))
            return tuple(out)

        vmaxes = lax.fori_loop(0, CHUNKS // UZ, amax_body,
                               (_splat(-inf),) * UZ)
        vm = vmaxes[0]
        for u in range(1, UZ):
            vm = jnp.maximum(vm, vmaxes[u])
        vs_sorted, _ = plsc.sort_key_val(vm, vm)
        zero_pad = jnp.where(lane_iota >= L - KTOP + 1, -inf, vs_sorted)
        t_cand = jnp.max(zero_pad)
        t_cand_v = _splat(t_cand)

        def grp_body(g, carry):
            Cv, Ci = carry
            base = g * (GRP * L)
            vs = [row_v[pl.ds(base + c * L, L)] for c in range(GRP)]
            ges = [v >= t_cand_v for v in vs]
            any_ge = ges[0]
            for c in range(1, GRP):
                any_ge = jnp.logical_or(any_ge, ges[c])
            pred = jnp.any(any_ge)

            def merge(carry):
                Cv, Ci = carry
                for c in range(GRP):
                    def merge_chunk(carry, c=c):
                        Cv, Ci = carry
                        idxv = _splat(base + c * L, jnp.int32) + lane_iota
                        sv, si = plsc.sort_key_val(vs[c], idxv,
                                                   descending=True)
                        take = sv > Cv
                        nCv = jnp.where(take, sv, Cv)
                        nCi = jnp.where(take, si, Ci)
                        nv, ni = plsc.sort_key_val(nCv, nCi)
                        return nv, ni

                    Cv, Ci = lax.cond(jnp.any(ges[c]), merge_chunk,
                                      lambda c_: c_, (Cv, Ci))
                return Cv, Ci

            return lax.cond(pred, merge, lambda c_: c_, (Cv, Ci))

        Cv, Ci = lax.fori_loop(
            0, NGRP, grp_body,
            (_splat(-inf), _splat(0, jnp.int32)))

        t10 = jnp.max(jnp.where(lane_iota >= L - KTOP + 1, -inf, Cv))
        t10v = _splat(t10)
        g_cnt = jnp.sum((Cv > t10v).astype(jnp.int32))
        r_take = KTOP - g_cnt
        eq_cnt = jnp.sum((Cv == t10v).astype(jnp.int32))
        unambiguous = eq_cnt == r_take

        def write_common(_):
            keep = Cv >= t10v
            plsc.store_scatter(zero_v, [Ci], ones_f, mask=keep)
            pltpu.sync_copy(zero_v, out_hbm.at[row])
            plsc.store_scatter(zero_v, [Ci], zeros_f, mask=keep)
            return 0

        def write_fallback(_):
            def wgrp(g, e_seen):
                base = g * (GRP * L)
                e = e_seen
                for c in range(GRP):
                    v = row_v[pl.ds(base + c * L, L)]
                    eq = v == t10v
                    eq_i = eq.astype(jnp.int32)
                    prefix = plsc.cumsum(eq_i) + e
                    sel = jnp.logical_and(eq, prefix <= r_take)
                    m = jnp.logical_or(v > t10v, sel)
                    zero_v[pl.ds(base + c * L, L)] = jnp.where(m, 1.0, 0.0)
                    e = e + jnp.sum(eq_i)
                return e

            lax.fori_loop(0, NGRP, wgrp, jnp.int32(0))
            pltpu.sync_copy(zero_v, out_hbm.at[row])
            lax.fori_loop(0, CHUNKS // UZ, zfill, 0)
            return 0

        lax.cond(unambiguous, write_common, write_fallback, 0)

        if rr + 1 < ROWS_PER_W:
            nxt.wait()


@jax.jit
def kernel(scores):
    mesh = plsc.VectorSubcoreMesh(core_axis_name="c", subcore_axis_name="s")
    sc_call = functools.partial(
        pl.kernel,
        out_type=jax.ShapeDtypeStruct((B, N), jnp.float32),
        mesh=mesh,
        compiler_params=pltpu.CompilerParams(needs_layout_passes=False),
        scratch_types=[
            pltpu.VMEM((N,), jnp.float32),
            pltpu.VMEM((N,), jnp.float32),
            pltpu.VMEM((N,), jnp.float32),
            pltpu.SemaphoreType.DMA,
            pltpu.SemaphoreType.DMA,
        ],
    )(_sc_body)
    return sc_call(scores)


# speedup vs baseline: 2.8705x; 1.0491x over previous
import functools

import jax
import jax.numpy as jnp
from jax import lax
from jax.experimental import pallas as pl
from jax.experimental.pallas import tpu as pltpu
from jax.experimental.pallas import tpu_sc as plsc

NC, NS, L = 2, 16, 16
NW = NC * NS
B, N = 128, 32768
ROWS_PER_W = B // NW
CHUNKS = N // L
GRP = 16
NGRP = CHUNKS // GRP
UZ = 8
UA, NACC = 32, 8
KTOP = 10


def _splat(x, dtype=jnp.float32):
    return jnp.full((L,), x, dtype)


def _sc_body(scores_hbm, out_hbm, row_a, row_b, zero_v, in_sem, out_sem):
    cid = lax.axis_index("c")
    sid = lax.axis_index("s")
    wid = sid * NC + cid
    inf = jnp.float32(jnp.inf)
    zeros_f = _splat(0.0)
    ones_f = _splat(1.0)
    lane_iota = lax.broadcasted_iota(jnp.int32, (L,), 0)

    def zfill(i, _):
        for u in range(UZ):
            zero_v[pl.ds((i * UZ + u) * L, L)] = zeros_f
        return 0

    lax.fori_loop(0, CHUNKS // UZ, zfill, 0)

    row0 = wid * ROWS_PER_W
    bufs = (row_a, row_b)
    pltpu.async_copy(scores_hbm.at[row0], row_a, in_sem).wait()

    pending = None
    for rr in range(ROWS_PER_W):
        row = row0 + rr
        row_v = bufs[rr % 2]
        if rr + 1 < ROWS_PER_W:
            nxt = pltpu.async_copy(scores_hbm.at[row + 1], bufs[(rr + 1) % 2],
                                   in_sem)

        def amax_body(i, vs):
            vs = list(vs)
            for u in range(UA):
                v = row_v[pl.ds((i * UA + u) * L, L)]
                vs[u % NACC] = jnp.maximum(vs[u % NACC], v)
            return tuple(vs)

        vmaxes = lax.fori_loop(0, CHUNKS // UA, amax_body,
                               (_splat(-inf),) * NACC)
        vm = vmaxes[0]
        for u in range(1, NACC):
            vm = jnp.maximum(vm, vmaxes[u])
        vs_sorted, _ = plsc.sort_key_val(vm, vm)
        t_cand = jnp.max(jnp.where(lane_iota >= L - KTOP + 1, -inf,
                                   vs_sorted))
        t_cand_v = _splat(t_cand)

        def grp_body(g, carry):
            Cv, Ci = carry
            base = g * (GRP * L)
            vs = [row_v[pl.ds(base + c * L, L)] for c in range(GRP)]
            ges = [v >= t_cand_v for v in vs]
            any_ge = ges[0]
            for c in range(1, GRP):
                any_ge = jnp.logical_or(any_ge, ges[c])
            pred = jnp.any(any_ge)

            def merge(carry):
                Cv, Ci = carry
                for c in range(GRP):
                    def merge_chunk(carry, c=c):
                        Cv, Ci = carry
                        idxv = _splat(base + c * L, jnp.int32) + lane_iota
                        sv, si = plsc.sort_key_val(vs[c], idxv,
                                                   descending=True)
                        take = sv > Cv
                        nCv = jnp.where(take, sv, Cv)
                        nCi = jnp.where(take, si, Ci)
                        nv, ni = plsc.sort_key_val(nCv, nCi)
                        return nv, ni

                    Cv, Ci = lax.cond(jnp.any(ges[c]), merge_chunk,
                                      lambda c_: c_, (Cv, Ci))
                return Cv, Ci

            return lax.cond(pred, merge, lambda c_: c_, (Cv, Ci))

        Cv, Ci = lax.fori_loop(
            0, NGRP, grp_body,
            (_splat(-inf), _splat(0, jnp.int32)))

        t10 = jnp.max(jnp.where(lane_iota >= L - KTOP + 1, -inf, Cv))
        t10v = _splat(t10)
        g_cnt = jnp.sum((Cv > t10v).astype(jnp.int32))
        r_take = KTOP - g_cnt
        eq_cnt = jnp.sum((Cv == t10v).astype(jnp.int32))
        unambiguous = eq_cnt == r_take
        keep = Cv >= t10v

        if pending is not None:
            p_keep, p_ci, p_unamb = pending
            pltpu.make_async_copy(zero_v, out_hbm.at[row - 1], out_sem).wait()

            def rest_common(_):
                plsc.store_scatter(zero_v, [p_ci], zeros_f, mask=p_keep)
                return 0

            def rest_fallback(_):
                lax.fori_loop(0, CHUNKS // UZ, zfill, 0)
                return 0

            lax.cond(p_unamb, rest_common, rest_fallback, 0)

        def write_common(_):
            plsc.store_scatter(zero_v, [Ci], ones_f, mask=keep)
            return 0

        def write_fallback(_):
            def wgrp(g, e_seen):
                base = g * (GRP * L)
                e = e_seen
                for c in range(GRP):
                    v = row_v[pl.ds(base + c * L, L)]
                    eq = v == t10v
                    eq_i = eq.astype(jnp.int32)
                    prefix = plsc.cumsum(eq_i) + e
                    sel = jnp.logical_and(eq, prefix <= r_take)
                    m = jnp.logical_or(v > t10v, sel)
                    zero_v[pl.ds(base + c * L, L)] = jnp.where(m, 1.0, 0.0)
                    e = e + jnp.sum(eq_i)
                return e

            lax.fori_loop(0, NGRP, wgrp, jnp.int32(0))
            return 0

        lax.cond(unambiguous, write_common, write_fallback, 0)
        pltpu.async_copy(zero_v, out_hbm.at[row], out_sem)
        pending = (keep, Ci, unambiguous)

        if rr + 1 < ROWS_PER_W:
            nxt.wait()

    pltpu.make_async_copy(zero_v, out_hbm.at[row0 + ROWS_PER_W - 1],
                          out_sem).wait()


@jax.jit
def kernel(scores):
    mesh = plsc.VectorSubcoreMesh(core_axis_name="c", subcore_axis_name="s")
    sc_call = functools.partial(
        pl.kernel,
        out_type=jax.ShapeDtypeStruct((B, N), jnp.float32),
        mesh=mesh,
        compiler_params=pltpu.CompilerParams(needs_layout_passes=False),
        scratch_types=[
            pltpu.VMEM((N,), jnp.float32),
            pltpu.VMEM((N,), jnp.float32),
            pltpu.VMEM((N,), jnp.float32),
            pltpu.SemaphoreType.DMA,
            pltpu.SemaphoreType.DMA,
        ],
    )(_sc_body)
    return sc_call(scores)


# speedup vs baseline: 3.0951x; 1.0783x over previous
import functools

import jax
import jax.numpy as jnp
from jax import lax
from jax.experimental import pallas as pl
from jax.experimental.pallas import tpu as pltpu
from jax.experimental.pallas import tpu_sc as plsc

NC, NS, L = 2, 16, 16
NW = NC * NS
B, N = 128, 32768
ROWS_PER_W = B // NW
CHUNKS = N // L
GRP = 16
NGRP = CHUNKS // GRP
UZ = 8
SUB = 8
SGRP = 4
KTOP = 10


def _splat(x, dtype=jnp.float32):
    return jnp.full((L,), x, dtype)


def _sc_body(scores_hbm, out_hbm, row_a, row_b, zero_v, smax_v,
             in_sem, out_sem):
    cid = lax.axis_index("c")
    sid = lax.axis_index("s")
    wid = sid * NC + cid
    inf = jnp.float32(jnp.inf)
    zeros_f = _splat(0.0)
    ones_f = _splat(1.0)
    lane_iota = lax.broadcasted_iota(jnp.int32, (L,), 0)

    def zfill(i, _):
        for u in range(UZ):
            zero_v[pl.ds((i * UZ + u) * L, L)] = zeros_f
        return 0

    lax.fori_loop(0, CHUNKS // UZ, zfill, 0)

    row0 = wid * ROWS_PER_W
    bufs = (row_a, row_b)
    pltpu.async_copy(scores_hbm.at[row0], row_a, in_sem).wait()

    pending = None
    for rr in range(ROWS_PER_W):
        row = row0 + rr
        row_v = bufs[rr % 2]
        if rr + 1 < ROWS_PER_W:
            nxt = pltpu.async_copy(scores_hbm.at[row + 1], bufs[(rr + 1) % 2],
                                   in_sem)

        def amax_body(i, g_all):
            base = i * (SGRP * SUB * L)
            for s in range(SGRP):
                acc = row_v[pl.ds(base + s * SUB * L, L)]
                for u in range(1, SUB):
                    acc = jnp.maximum(
                        acc, row_v[pl.ds(base + (s * SUB + u) * L, L)])
                smax_v[pl.ds((i * SGRP + s) * L, L)] = acc
                g_all = jnp.maximum(g_all, acc)
            return g_all

        vm = lax.fori_loop(0, CHUNKS // (SGRP * SUB), amax_body,
                           _splat(-inf))
        vs_sorted, _ = plsc.sort_key_val(vm, vm)
        t_cand = jnp.max(jnp.where(lane_iota >= L - KTOP + 1, -inf,
                                   vs_sorted))
        t_cand_v = _splat(t_cand)

        def grp_body(g, carry):
            Cv, Ci = carry
            sm = [smax_v[pl.ds((g * SGRP + s) * L, L)] for s in range(SGRP)]
            gm = jnp.maximum(jnp.maximum(sm[0], sm[1]),
                             jnp.maximum(sm[2], sm[3]))
            pred = jnp.any(gm >= t_cand_v)

            def scan_group(carry):
                Cv, Ci = carry
                for s in range(SGRP):
                    def scan_sub(carry, s=s):
                        Cv, Ci = carry
                        base = (g * SGRP + s) * (SUB * L)
                        for c in range(SUB):
                            v = row_v[pl.ds(base + c * L, L)]

                            def merge_chunk(carry, v=v, c=c, base=base):
                                Cv, Ci = carry
                                idxv = (_splat(base + c * L, jnp.int32)
                                        + lane_iota)
                                sv, si = plsc.sort_key_val(v, idxv,
                                                           descending=True)
                                take = sv > Cv
                                nCv = jnp.where(take, sv, Cv)
                                nCi = jnp.where(take, si, Ci)
                                nv, ni = plsc.sort_key_val(nCv, nCi)
                                return nv, ni

                            Cv, Ci = lax.cond(jnp.any(v >= t_cand_v),
                                              merge_chunk, lambda c_: c_,
                                              (Cv, Ci))
                        return Cv, Ci

                    Cv, Ci = lax.cond(jnp.any(sm[s] >= t_cand_v), scan_sub,
                                      lambda c_: c_, (Cv, Ci))
                return Cv, Ci

            return lax.cond(pred, scan_group, lambda c_: c_, (Cv, Ci))

        Cv, Ci = lax.fori_loop(
            0, CHUNKS // (SGRP * SUB), grp_body,
            (_splat(-inf), _splat(0, jnp.int32)))

        t10 = jnp.max(jnp.where(lane_iota >= L - KTOP + 1, -inf, Cv))
        t10v = _splat(t10)
        g_cnt = jnp.sum((Cv > t10v).astype(jnp.int32))
        r_take = KTOP - g_cnt
        eq_cnt = jnp.sum((Cv == t10v).astype(jnp.int32))
        unambiguous = eq_cnt == r_take
        keep = Cv >= t10v

        if pending is not None:
            p_keep, p_ci, p_unamb = pending
            pltpu.make_async_copy(zero_v, out_hbm.at[row - 1], out_sem).wait()

            def rest_common(_):
                plsc.store_scatter(zero_v, [p_ci], zeros_f, mask=p_keep)
                return 0

            def rest_fallback(_):
                lax.fori_loop(0, CHUNKS // UZ, zfill, 0)
                return 0

            lax.cond(p_unamb, rest_common, rest_fallback, 0)

        def write_common(_):
            plsc.store_scatter(zero_v, [Ci], ones_f, mask=keep)
            return 0

        def write_fallback(_):
            def wgrp(g, e_seen):
                base = g * (GRP * L)
                e = e_seen
                for c in range(GRP):
                    v = row_v[pl.ds(base + c * L, L)]
                    eq = v == t10v
                    eq_i = eq.astype(jnp.int32)
                    prefix = plsc.cumsum(eq_i) + e
                    sel = jnp.logical_and(eq, prefix <= r_take)
                    m = jnp.logical_or(v > t10v, sel)
                    zero_v[pl.ds(base + c * L, L)] = jnp.where(m, 1.0, 0.0)
                    e = e + jnp.sum(eq_i)
                return e

            lax.fori_loop(0, NGRP, wgrp, jnp.int32(0))
            return 0

        lax.cond(unambiguous, write_common, write_fallback, 0)
        pltpu.async_copy(zero_v, out_hbm.at[row], out_sem)
        pending = (keep, Ci, unambiguous)

        if rr + 1 < ROWS_PER_W:
            nxt.wait()

    pltpu.make_async_copy(zero_v, out_hbm.at[row0 + ROWS_PER_W - 1],
                          out_sem).wait()


@jax.jit
def kernel(scores):
    mesh = plsc.VectorSubcoreMesh(core_axis_name="c", subcore_axis_name="s")
    sc_call = functools.partial(
        pl.kernel,
        out_type=jax.ShapeDtypeStruct((B, N), jnp.float32),
        mesh=mesh,
        compiler_params=pltpu.CompilerParams(needs_layout_passes=False),
        scratch_types=[
            pltpu.VMEM((N,), jnp.float32),
            pltpu.VMEM((N,), jnp.float32),
            pltpu.VMEM((N,), jnp.float32),
            pltpu.VMEM((N // SUB,), jnp.float32),
            pltpu.SemaphoreType.DMA,
            pltpu.SemaphoreType.DMA,
        ],
    )(_sc_body)
    return sc_call(scores)
